# Initial kernel scaffold; baseline (speedup 1.0000x reference)
import jax
import jax.numpy as jnp
from jax.experimental import pallas as pl


def kernel(x, lengths):
    raise NotImplementedError("write your pallas kernel here")


# speedup vs baseline: 2.1084x; 2.1084x over previous
import functools

import jax
import jax.numpy as jnp
from jax import lax
from jax.experimental import pallas as pl
from jax.experimental.pallas import tpu as pltpu

_K = 512
_NEG = float("-inf")


def _stage(x, d, run, lanes, flip=False):
    m = x.shape[0]
    run_shift = run.bit_length() - 1
    if d >= 8:
        k = m // (2 * d)
        xr = x.reshape(k, 2, d, lanes)
        a = xr[:, 0]
        b = xr[:, 1]
        mx = jnp.maximum(a, b)
        mn = jnp.minimum(a, b)
        shift = run_shift - (2 * d).bit_length() + 1
        blk = lax.broadcasted_iota(jnp.int32, (k, 1, 1), 0)
        desc = ((blk >> shift) & 1) == 0
        if flip is not False:
            desc = desc != flip
        lo = jnp.where(desc, mx, mn)
        hi = jnp.where(desc, mn, mx)
        return jnp.concatenate([lo[:, None], hi[:, None]], axis=1).reshape(m, lanes)
    else:
        i = lax.broadcasted_iota(jnp.int32, (m, 1), 0)
        desc = ((i >> run_shift) & 1) == 0
        if flip is not False:
            desc = desc != flip
        is_lower = (i & d) == 0
        partner = jnp.where(is_lower, jnp.roll(x, -d, axis=0), jnp.roll(x, d, axis=0))
        mx = jnp.maximum(x, partner)
        mn = jnp.minimum(x, partner)
        want_max = is_lower == desc
        return jnp.where(want_max, mx, mn)


def _sort_chunks(x, chunk, lanes):
    run = 2
    while run <= chunk:
        d = run // 2
        while d >= 1:
            x = _stage(x, d, run, lanes)
            d //= 2
        run *= 2
    return x


def _merge_tree(x, chunk, nchunks, lanes):
    n = nchunks // 2
    while n >= 1:
        xr = x.reshape(n, 2, chunk, lanes)
        x = jnp.maximum(xr[:, 0], xr[:, 1]).reshape(n * chunk, lanes)
        d = chunk // 2
        while d >= 1:
            x = _stage(x, d, chunk, lanes)
            d //= 2
        n //= 2
    return x


def _topk_body(len_ref, x_ref, o_ref, *, s, k, lanes):
    b = pl.program_id(0)
    length = len_ref[b]
    x = x_ref[0]
    row = lax.broadcasted_iota(jnp.int32, (s, 1), 0)
    x = jnp.where(row < length, x, _NEG)
    x = _sort_chunks(x, k, lanes)
    x = _merge_tree(x, k, s // k, lanes)
    newl = jnp.minimum(length, k)
    orow = lax.broadcasted_iota(jnp.int32, (k, 1), 0)
    o_ref[0] = jnp.where(orow < newl, x, 0.0)


def _build(s, d_total, k, lanes, interpret=False):
    def call(x, lengths):
        bsz = x.shape[0]
        body = functools.partial(_topk_body, s=s, k=k, lanes=lanes)
        return pl.pallas_call(
            body,
            grid=(bsz, d_total // lanes),
            in_specs=[
                pl.BlockSpec(memory_space=pltpu.SMEM),
                pl.BlockSpec((1, s, lanes), lambda b, dt: (b, 0, dt)),
            ],
            out_specs=pl.BlockSpec((1, k, lanes), lambda b, dt: (b, 0, dt)),
            out_shape=jax.ShapeDtypeStruct((bsz, k, d_total), jnp.float32),
            compiler_params=pltpu.CompilerParams(
                dimension_semantics=("parallel", "parallel"),
            ),
            interpret=interpret,
        )(lengths, x)

    return call


@jax.jit
def kernel(x, lengths):
    bsz, s, d_total = x.shape
    pooled = _build(s, d_total, _K, 128)(x, lengths)
    return pooled, jnp.minimum(lengths, _K)


# speedup vs baseline: 5.8139x; 2.7574x over previous
import functools

import jax
import jax.numpy as jnp
from jax import lax
from jax.experimental import pallas as pl
from jax.experimental.pallas import tpu as pltpu

_K = 512
_NEG = float("-inf")


def _stage(x, d, run, lanes, flip=False):
    m = x.shape[0]
    run_shift = run.bit_length() - 1
    if d >= 8:
        k = m // (2 * d)
        xr = x.reshape(k, 2, d, lanes)
        a = xr[:, 0]
        b = xr[:, 1]
        mx = jnp.maximum(a, b)
        mn = jnp.minimum(a, b)
        shift = run_shift - (2 * d).bit_length() + 1
        blk = lax.broadcasted_iota(jnp.int32, (k, 1, 1), 0)
        desc = ((blk >> shift) & 1) == 0
        if flip is not False:
            desc = desc != flip
        lo = jnp.where(desc, mx, mn)
        hi = jnp.where(desc, mn, mx)
        return jnp.concatenate([lo[:, None], hi[:, None]], axis=1).reshape(m, lanes)
    else:
        i = lax.broadcasted_iota(jnp.int32, (m, 1), 0)
        desc = ((i >> run_shift) & 1) == 0
        if flip is not False:
            desc = desc != flip
        is_lower = (i & d) == 0
        partner = jnp.where(is_lower, jnp.roll(x, -d, axis=0), jnp.roll(x, d, axis=0))
        mx = jnp.maximum(x, partner)
        mn = jnp.minimum(x, partner)
        want_max = is_lower == desc
        return jnp.where(want_max, mx, mn)


def _sort_chunks(x, chunk, lanes, flip=False):
    run = 2
    while run <= chunk:
        d = run // 2
        while d >= 1:
            x = _stage(x, d, run, lanes, flip)
            d //= 2
        run *= 2
    return x


def _merge_tree(x, chunk, nchunks, lanes):
    n = nchunks // 2
    while n >= 1:
        xr = x.reshape(n, 2, chunk, lanes)
        x = jnp.maximum(xr[:, 0], xr[:, 1]).reshape(n * chunk, lanes)
        d = chunk // 2
        while d >= 1:
            x = _stage(x, d, chunk, lanes)
            d //= 2
        n //= 2
    return x


def _topk_body(len_ref, x_ref, o_ref, scratch, *, s, k, lanes):
    b = pl.program_id(0)
    length = len_ref[b]
    x = x_ref[0]
    row = lax.broadcasted_iota(jnp.int32, (s, 1), 0)
    scratch[:] = jnp.where(row < length, x, _NEG)

    nact = (length + (k - 1)) // k

    def chunk_body(c, carry):
        ch = scratch[pl.ds(c * k, k), :]
        ch = _sort_chunks(ch, k, lanes, flip=(c & 1) == 1)
        scratch[pl.ds(c * k, k), :] = ch
        return carry

    lax.fori_loop(0, nact, chunk_body, 0)
    y = _merge_tree(scratch[:], k, s // k, lanes)
    newl = jnp.minimum(length, k)
    orow = lax.broadcasted_iota(jnp.int32, (k, 1), 0)
    o_ref[0] = jnp.where(orow < newl, y, 0.0)


def _build(s, d_total, k, lanes, interpret=False):
    def call(x, lengths):
        bsz = x.shape[0]
        body = functools.partial(_topk_body, s=s, k=k, lanes=lanes)
        return pl.pallas_call(
            body,
            grid=(bsz, d_total // lanes),
            in_specs=[
                pl.BlockSpec(memory_space=pltpu.SMEM),
                pl.BlockSpec((1, s, lanes), lambda b, dt: (b, 0, dt)),
            ],
            out_specs=pl.BlockSpec((1, k, lanes), lambda b, dt: (b, 0, dt)),
            out_shape=jax.ShapeDtypeStruct((bsz, k, d_total), jnp.float32),
            scratch_shapes=[pltpu.VMEM((s, lanes), jnp.float32)],
            compiler_params=pltpu.CompilerParams(
                dimension_semantics=("parallel", "parallel"),
            ),
            interpret=interpret,
        )(lengths, x)

    return call


@jax.jit
def kernel(x, lengths):
    bsz, s, d_total = x.shape
    pooled = _build(s, d_total, _K, 128)(x, lengths)
    return pooled, jnp.minimum(lengths, _K)
